# baseline (speedup 1.0000x reference)
import functools

import jax
import jax.numpy as jnp
from jax import lax
from jax.experimental import pallas as pl
from jax.experimental.pallas import tpu as pltpu
from jax.experimental.pallas import tpu_sc as plsc

B = 16384
D = 32
GROUP = 128
RPG = GROUP // D
NROWS = 1000000
NG = NROWS // RPG
NC = 2
NS = 16
NW = NC * NS
BPW = B // NW
CHUNK = 128
NCHUNK = BPW // CHUNK


def _sc_gather_body(u_tab, i_tab, u_idx, i_idx, u_out, i_out,
                    idx_u_v, idx_i_v, rows_v, sem):
    wid = lax.axis_index("s") * NC + lax.axis_index("c")
    base = wid * BPW
    pltpu.sync_copy(u_idx.at[pl.ds(wid * NCHUNK, NCHUNK)], idx_u_v)
    pltpu.sync_copy(i_idx.at[pl.ds(wid * NCHUNK, NCHUNK)], idx_i_v)
    for j in range(NCHUNK):
        pltpu.async_copy(u_tab.at[idx_u_v.at[j]],
                         rows_v.at[pl.ds(j * CHUNK, CHUNK)], sem)
    for j in range(NCHUNK):
        pltpu.make_async_copy(u_tab.at[idx_u_v.at[j]],
                              rows_v.at[pl.ds(j * CHUNK, CHUNK)], sem).wait()
    pltpu.sync_copy(rows_v, u_out.at[pl.ds(base, BPW)])
    for j in range(NCHUNK):
        pltpu.async_copy(i_tab.at[idx_i_v.at[j]],
                         rows_v.at[pl.ds(j * CHUNK, CHUNK)], sem)
    for j in range(NCHUNK):
        pltpu.make_async_copy(i_tab.at[idx_i_v.at[j]],
                              rows_v.at[pl.ds(j * CHUNK, CHUNK)], sem).wait()
    pltpu.sync_copy(rows_v, i_out.at[pl.ds(base, BPW)])


_sc_gather = pl.kernel(
    _sc_gather_body,
    out_type=(jax.ShapeDtypeStruct((B, GROUP), jnp.float32),
              jax.ShapeDtypeStruct((B, GROUP), jnp.float32)),
    mesh=plsc.VectorSubcoreMesh(core_axis_name="c", subcore_axis_name="s",
                                num_cores=NC, num_subcores=NS),
    scratch_types=[
        pltpu.VMEM((NCHUNK, CHUNK), jnp.int32),
        pltpu.VMEM((NCHUNK, CHUNK), jnp.int32),
        pltpu.VMEM((BPW, GROUP), jnp.float32),
        pltpu.SemaphoreType.DMA,
    ],
)


BB = 2048


def _dense_body(gu, gi, uidc, iidc, uf, itf, wuf, wif, buf, bif, wout, bo,
                out):
    wu = wout[:, :D]
    wi = wout[:, D:]
    lane = lax.broadcasted_iota(jnp.int32, (RPG, GROUP), 1)
    block = lax.broadcasted_iota(jnp.int32, (RPG, GROUP), 0)
    wu_rep = jnp.tile(wu, (RPG, RPG))
    wi_rep = jnp.tile(wi, (RPG, RPG))
    keep = (lane // D) == block
    wu_shift = jnp.where(keep, wu_rep, 0.0)
    wi_shift = jnp.where(keep, wi_rep, 0.0)

    sel4 = lax.broadcasted_iota(jnp.int32, (BB, RPG), 1)
    oh_u = (uidc[...] % RPG == sel4).astype(jnp.float32)
    oh_i = (iidc[...] % RPG == sel4).astype(jnp.float32)
    row_wu = jnp.dot(oh_u, wu_shift, preferred_element_type=jnp.float32)
    row_wi = jnp.dot(oh_i, wi_shift, preferred_element_type=jnp.float32)

    ufe = jnp.dot(uf[...], wuf[...], preferred_element_type=jnp.float32)
    ife = jnp.dot(itf[...], wif[...], preferred_element_type=jnp.float32)
    const = (jnp.sum(buf[...] * wu) + jnp.sum(bif[...] * wi) + bo[0, 0])
    out[...] = (jnp.sum(gu[...] * row_wu, axis=1, keepdims=True)
                + jnp.sum(gi[...] * row_wi, axis=1, keepdims=True)
                + jnp.sum(ufe * wu, axis=1, keepdims=True)
                + jnp.sum(ife * wi, axis=1, keepdims=True)
                + const)


_dense = pl.pallas_call(
    _dense_body,
    grid=(B // BB,),
    in_specs=[
        pl.BlockSpec((BB, GROUP), lambda i: (i, 0)),
        pl.BlockSpec((BB, GROUP), lambda i: (i, 0)),
        pl.BlockSpec((BB, 1), lambda i: (i, 0)),
        pl.BlockSpec((BB, 1), lambda i: (i, 0)),
        pl.BlockSpec((BB, 16), lambda i: (i, 0)),
        pl.BlockSpec((BB, 16), lambda i: (i, 0)),
        pl.BlockSpec((16, D), lambda i: (0, 0)),
        pl.BlockSpec((16, D), lambda i: (0, 0)),
        pl.BlockSpec((1, D), lambda i: (0, 0)),
        pl.BlockSpec((1, D), lambda i: (0, 0)),
        pl.BlockSpec((1, 2 * D), lambda i: (0, 0)),
        pl.BlockSpec((1, 1), lambda i: (0, 0)),
    ],
    out_specs=pl.BlockSpec((BB, 1), lambda i: (i, 0)),
    out_shape=jax.ShapeDtypeStruct((B, 1), jnp.float32),
)


def kernel(user_ids, item_ids, user_features, item_features, user_emb,
           item_emb, W_uf, b_uf, W_if, b_if, W_out, b_out):
    u_tab = user_emb.reshape(NG, GROUP)
    i_tab = item_emb.reshape(NG, GROUP)
    uidx = (user_ids // RPG).reshape(NW * NCHUNK, CHUNK)
    iidx = (item_ids // RPG).reshape(NW * NCHUNK, CHUNK)
    gu, gi = _sc_gather(u_tab, i_tab, uidx, iidx)
    return _dense(gu, gi, user_ids.reshape(B, 1), item_ids.reshape(B, 1),
                  user_features, item_features, W_uf, W_if,
                  b_uf.reshape(1, D), b_if.reshape(1, D),
                  W_out.reshape(1, 2 * D), b_out.reshape(1, 1))


# speedup vs baseline: 4.8423x; 4.8423x over previous
import jax
import jax.numpy as jnp
from jax import lax
from jax.experimental import pallas as pl
from jax.experimental.pallas import tpu as pltpu
from jax.experimental.pallas import tpu_sc as plsc

B = 16384
D = 32
FD = 16
NROWS = 1000000
PADN = 1007616
GW = 128
PNG = PADN // GW
BLK = 8192
GA = PADN // BLK

NC = 2
NS = 16
NW = NC * NS
BPW = B // NW
CHUNK = 128
NCHUNK = BPW // CHUNK


def _pv_body(wout, ut, it, pu, pi):
    wu = wout[:, :D]
    wi = wout[:, D:]
    pu[...] = jnp.dot(wu, ut[...], preferred_element_type=jnp.float32
                      ).reshape(BLK)
    pi[...] = jnp.dot(wi, it[...], preferred_element_type=jnp.float32
                      ).reshape(BLK)


_pv = pl.pallas_call(
    _pv_body,
    grid=(GA,),
    in_specs=[
        pl.BlockSpec((1, 2 * D), lambda j: (0, 0)),
        pl.BlockSpec((D, BLK), lambda j: (0, j)),
        pl.BlockSpec((D, BLK), lambda j: (0, j)),
    ],
    out_specs=[
        pl.BlockSpec((BLK,), lambda j: (j,)),
        pl.BlockSpec((BLK,), lambda j: (j,)),
    ],
    out_shape=[
        jax.ShapeDtypeStruct((PADN,), jnp.float32),
        jax.ShapeDtypeStruct((PADN,), jnp.float32),
    ],
)


def _gather_one(tab, idx, out, idxv, rows, sem, wid):
    base = wid * BPW
    pltpu.sync_copy(idx.at[pl.ds(wid * NCHUNK, NCHUNK)], idxv)
    for j in range(NCHUNK):
        pltpu.async_copy(tab.at[idxv.at[j]],
                         rows.at[pl.ds(j * CHUNK, CHUNK)], sem)
    for j in range(NCHUNK):
        pltpu.make_async_copy(tab.at[idxv.at[j]],
                              rows.at[pl.ds(j * CHUNK, CHUNK)], sem).wait()
    pltpu.sync_copy(rows, out.at[pl.ds(base, BPW)])


def _sc_body(pu2, pi2, idxu, idxi, gu, gi, idxv, rows, sem):
    wid = lax.axis_index("s") * NC + lax.axis_index("c")
    _gather_one(pu2, idxu, gu, idxv, rows, sem, wid)
    _gather_one(pi2, idxi, gi, idxv, rows, sem, wid)


_sc_gather = pl.kernel(
    _sc_body,
    out_type=(jax.ShapeDtypeStruct((B, GW), jnp.float32),
              jax.ShapeDtypeStruct((B, GW), jnp.float32)),
    mesh=plsc.VectorSubcoreMesh(core_axis_name="c", subcore_axis_name="s",
                                num_cores=NC, num_subcores=NS),
    scratch_types=[
        pltpu.VMEM((NCHUNK, CHUNK), jnp.int32),
        pltpu.VMEM((BPW, GW), jnp.float32),
        pltpu.SemaphoreType.DMA,
    ],
)


BB = 4096


def _comb_body(gu, gi, uid, iid, uf, itf, wuf, wif, buf, bif, wout, bo, out):
    f32 = jnp.float32
    wu = wout[:, :D]
    wi = wout[:, D:]
    qu = lax.dot_general(wuf[...], wu, (((1,), (1,)), ((), ())),
                         preferred_element_type=f32)
    qi = lax.dot_general(wif[...], wi, (((1,), (1,)), ((), ())),
                         preferred_element_type=f32)
    lane = lax.broadcasted_iota(jnp.int32, (BB, GW), 1)
    ohu = (jnp.bitwise_and(uid[...], GW - 1) == lane).astype(f32)
    ohi = (jnp.bitwise_and(iid[...], GW - 1) == lane).astype(f32)
    selu = jnp.sum(gu[...] * ohu, axis=1, keepdims=True)
    seli = jnp.sum(gi[...] * ohi, axis=1, keepdims=True)
    fu = jnp.dot(uf[...], qu, preferred_element_type=f32)
    fi = jnp.dot(itf[...], qi, preferred_element_type=f32)
    const = (jnp.sum(buf[...] * wu) + jnp.sum(bif[...] * wi) + bo[0, 0])
    out[...] = selu + seli + fu + fi + const


_combine = pl.pallas_call(
    _comb_body,
    grid=(B // BB,),
    in_specs=[
        pl.BlockSpec((BB, GW), lambda i: (i, 0)),
        pl.BlockSpec((BB, GW), lambda i: (i, 0)),
        pl.BlockSpec((BB, 1), lambda i: (i, 0)),
        pl.BlockSpec((BB, 1), lambda i: (i, 0)),
        pl.BlockSpec((BB, FD), lambda i: (i, 0)),
        pl.BlockSpec((BB, FD), lambda i: (i, 0)),
        pl.BlockSpec((FD, D), lambda i: (0, 0)),
        pl.BlockSpec((FD, D), lambda i: (0, 0)),
        pl.BlockSpec((1, D), lambda i: (0, 0)),
        pl.BlockSpec((1, D), lambda i: (0, 0)),
        pl.BlockSpec((1, 2 * D), lambda i: (0, 0)),
        pl.BlockSpec((1, 1), lambda i: (0, 0)),
    ],
    out_specs=pl.BlockSpec((BB, 1), lambda i: (i, 0)),
    out_shape=jax.ShapeDtypeStruct((B, 1), jnp.float32),
)


def kernel(user_ids, item_ids, user_features, item_features, user_emb,
           item_emb, W_uf, b_uf, W_if, b_if, W_out, b_out):
    wout = W_out.reshape(1, 2 * D)
    pu, pi = _pv(wout, user_emb.T, item_emb.T)
    idxu = lax.shift_right_logical(user_ids, 7).reshape(NW * NCHUNK, CHUNK)
    idxi = lax.shift_right_logical(item_ids, 7).reshape(NW * NCHUNK, CHUNK)
    gu, gi = _sc_gather(pu.reshape(PNG, GW), pi.reshape(PNG, GW),
                        idxu, idxi)
    return _combine(gu, gi, user_ids.reshape(B, 1), item_ids.reshape(B, 1),
                    user_features, item_features, W_uf, W_if,
                    b_uf.reshape(1, D), b_if.reshape(1, D), wout,
                    b_out.reshape(1, 1))


# speedup vs baseline: 8.3313x; 1.7205x over previous
import jax
import jax.numpy as jnp
from jax import lax
from jax.experimental import pallas as pl
from jax.experimental.pallas import tpu as pltpu
from jax.experimental.pallas import tpu_sc as plsc

B = 16384
D = 32
FD = 16
NROWS = 1000000
BLK = 16384
PADN = 1015808
GA = PADN // BLK

NC = 2
NS = 16
NW = NC * NS
BPW = B // NW
CHUNK = 128
NCHUNK = BPW // CHUNK


def _pv_body(wout, ut, it, pu, pi):
    wu = wout[:, :D]
    wi = wout[:, D:]
    pu[...] = jnp.dot(wu, ut[...], preferred_element_type=jnp.float32
                      ).reshape(BLK)
    pi[...] = jnp.dot(wi, it[...], preferred_element_type=jnp.float32
                      ).reshape(BLK)


_pv = pl.pallas_call(
    _pv_body,
    grid=(GA,),
    in_specs=[
        pl.BlockSpec((1, 2 * D), lambda j: (0, 0)),
        pl.BlockSpec((D, BLK), lambda j: (0, j)),
        pl.BlockSpec((D, BLK), lambda j: (0, j)),
    ],
    out_specs=[
        pl.BlockSpec((BLK,), lambda j: (j,)),
        pl.BlockSpec((BLK,), lambda j: (j,)),
    ],
    out_shape=[
        jax.ShapeDtypeStruct((PADN,), jnp.float32),
        jax.ShapeDtypeStruct((PADN,), jnp.float32),
    ],
)


def _gather_one(tab, idx, out, idxv, vals, sem, wid):
    base = wid * BPW
    pltpu.sync_copy(idx.at[pl.ds(wid * NCHUNK, NCHUNK)], idxv)
    for j in range(NCHUNK):
        pltpu.async_copy(tab.at[idxv.at[j]],
                         vals.at[pl.ds(j * CHUNK, CHUNK)], sem)
    for j in range(NCHUNK):
        pltpu.make_async_copy(tab.at[idxv.at[j]],
                              vals.at[pl.ds(j * CHUNK, CHUNK)], sem).wait()
    pltpu.sync_copy(vals, out.at[pl.ds(base, BPW)])


def _sc_body(pu, pi, idxu, idxi, su, si, idxv, vals, sem):
    wid = lax.axis_index("s") * NC + lax.axis_index("c")
    _gather_one(pu, idxu, su, idxv, vals, sem, wid)
    _gather_one(pi, idxi, si, idxv, vals, sem, wid)


_sc_gather = pl.kernel(
    _sc_body,
    out_type=(jax.ShapeDtypeStruct((B,), jnp.float32),
              jax.ShapeDtypeStruct((B,), jnp.float32)),
    mesh=plsc.VectorSubcoreMesh(core_axis_name="c", subcore_axis_name="s",
                                num_cores=NC, num_subcores=NS),
    scratch_types=[
        pltpu.VMEM((NCHUNK, CHUNK), jnp.int32),
        pltpu.VMEM((BPW,), jnp.float32),
        pltpu.SemaphoreType.DMA,
    ],
)


BB = 4096


def _comb_body(su, si, uft, ift, wuf, wif, buf, bif, wout, bo, out):
    f32 = jnp.float32
    wu = wout[:, :D]
    wi = wout[:, D:]
    qu = lax.dot_general(wuf[...], wu, (((1,), (1,)), ((), ())),
                         preferred_element_type=f32)
    qi = lax.dot_general(wif[...], wi, (((1,), (1,)), ((), ())),
                         preferred_element_type=f32)
    fu = lax.dot_general(qu, uft[...], (((0,), (0,)), ((), ())),
                         preferred_element_type=f32)
    fi = lax.dot_general(qi, ift[...], (((0,), (0,)), ((), ())),
                         preferred_element_type=f32)
    const = (jnp.sum(buf[...] * wu) + jnp.sum(bif[...] * wi) + bo[0, 0])
    out[...] = su[...] + si[...] + fu + fi + const


_combine = pl.pallas_call(
    _comb_body,
    grid=(B // BB,),
    in_specs=[
        pl.BlockSpec((1, BB), lambda i: (0, i)),
        pl.BlockSpec((1, BB), lambda i: (0, i)),
        pl.BlockSpec((FD, BB), lambda i: (0, i)),
        pl.BlockSpec((FD, BB), lambda i: (0, i)),
        pl.BlockSpec((FD, D), lambda i: (0, 0)),
        pl.BlockSpec((FD, D), lambda i: (0, 0)),
        pl.BlockSpec((1, D), lambda i: (0, 0)),
        pl.BlockSpec((1, D), lambda i: (0, 0)),
        pl.BlockSpec((1, 2 * D), lambda i: (0, 0)),
        pl.BlockSpec((1, 1), lambda i: (0, 0)),
    ],
    out_specs=pl.BlockSpec((1, BB), lambda i: (0, i)),
    out_shape=jax.ShapeDtypeStruct((1, B), jnp.float32),
)


def kernel(user_ids, item_ids, user_features, item_features, user_emb,
           item_emb, W_uf, b_uf, W_if, b_if, W_out, b_out):
    wout = W_out.reshape(1, 2 * D)
    pu, pi = _pv(wout, user_emb.T, item_emb.T)
    idxu = user_ids.reshape(NW * NCHUNK, CHUNK)
    idxi = item_ids.reshape(NW * NCHUNK, CHUNK)
    su, si = _sc_gather(pu, pi, idxu, idxi)
    out = _combine(su.reshape(1, B), si.reshape(1, B),
                   user_features.T, item_features.T, W_uf, W_if,
                   b_uf.reshape(1, D), b_if.reshape(1, D), wout,
                   b_out.reshape(1, 1))
    return out.reshape(B, 1)


# speedup vs baseline: 8.5398x; 1.0250x over previous
import jax
import jax.numpy as jnp
from jax import lax
from jax.experimental import pallas as pl
from jax.experimental.pallas import tpu as pltpu
from jax.experimental.pallas import tpu_sc as plsc

B = 16384
D = 32
FD = 16
NROWS = 1000000
BLK = 16384
PADN = 1015808
GA = PADN // BLK

NC = 2
NS = 16
NW = NC * NS
BPW = B // NW
CHUNK = 128
NCHUNK = BPW // CHUNK


def _pv_body(wout, ut, it, uft, ift, wuf, wif, buf, bif, bo, pu, pi, f):
    f32 = jnp.float32
    wu = wout[:, :D]
    wi = wout[:, D:]
    pu[...] = jnp.dot(wu, ut[...], preferred_element_type=f32).reshape(BLK)
    pi[...] = jnp.dot(wi, it[...], preferred_element_type=f32).reshape(BLK)

    @pl.when(pl.program_id(0) == 0)
    def _():
        qu = lax.dot_general(wuf[...], wu, (((1,), (1,)), ((), ())),
                             preferred_element_type=f32)
        qi = lax.dot_general(wif[...], wi, (((1,), (1,)), ((), ())),
                             preferred_element_type=f32)
        fu = lax.dot_general(qu, uft[...], (((0,), (0,)), ((), ())),
                             preferred_element_type=f32)
        fi = lax.dot_general(qi, ift[...], (((0,), (0,)), ((), ())),
                             preferred_element_type=f32)
        const = (jnp.sum(buf[...] * wu) + jnp.sum(bif[...] * wi) + bo[0, 0])
        f[...] = fu + fi + const


_pv = pl.pallas_call(
    _pv_body,
    grid=(GA,),
    in_specs=[
        pl.BlockSpec((1, 2 * D), lambda j: (0, 0)),
        pl.BlockSpec((D, BLK), lambda j: (0, j)),
        pl.BlockSpec((D, BLK), lambda j: (0, j)),
        pl.BlockSpec((FD, B), lambda j: (0, 0)),
        pl.BlockSpec((FD, B), lambda j: (0, 0)),
        pl.BlockSpec((FD, D), lambda j: (0, 0)),
        pl.BlockSpec((FD, D), lambda j: (0, 0)),
        pl.BlockSpec((1, D), lambda j: (0, 0)),
        pl.BlockSpec((1, D), lambda j: (0, 0)),
        pl.BlockSpec((1, 1), lambda j: (0, 0)),
    ],
    out_specs=[
        pl.BlockSpec((BLK,), lambda j: (j,)),
        pl.BlockSpec((BLK,), lambda j: (j,)),
        pl.BlockSpec((1, B), lambda j: (0, 0)),
    ],
    out_shape=[
        jax.ShapeDtypeStruct((PADN,), jnp.float32),
        jax.ShapeDtypeStruct((PADN,), jnp.float32),
        jax.ShapeDtypeStruct((1, B), jnp.float32),
    ],
)


def _sc_body(pu, pi, f, idxu, idxi, out, idxvu, idxvi, valsu, valsi, fv,
             sem):
    wid = lax.axis_index("s") * NC + lax.axis_index("c")
    base = wid * BPW
    pltpu.sync_copy(idxu.at[pl.ds(wid * NCHUNK, NCHUNK)], idxvu)
    pltpu.sync_copy(idxi.at[pl.ds(wid * NCHUNK, NCHUNK)], idxvi)
    for j in range(NCHUNK):
        pltpu.async_copy(pu.at[idxvu.at[j]],
                         valsu.at[pl.ds(j * CHUNK, CHUNK)], sem)
    for j in range(NCHUNK):
        pltpu.async_copy(pi.at[idxvi.at[j]],
                         valsi.at[pl.ds(j * CHUNK, CHUNK)], sem)
    pltpu.sync_copy(f.at[pl.ds(base, BPW)], fv)
    for j in range(NCHUNK):
        pltpu.make_async_copy(pu.at[idxvu.at[j]],
                              valsu.at[pl.ds(j * CHUNK, CHUNK)], sem).wait()
    for j in range(NCHUNK):
        pltpu.make_async_copy(pi.at[idxvi.at[j]],
                              valsi.at[pl.ds(j * CHUNK, CHUNK)], sem).wait()
    valsu[...] = valsu[...] + valsi[...] + fv[...]
    pltpu.sync_copy(valsu, out.at[pl.ds(base, BPW)])


_sc_gather = pl.kernel(
    _sc_body,
    out_type=jax.ShapeDtypeStruct((B,), jnp.float32),
    mesh=plsc.VectorSubcoreMesh(core_axis_name="c", subcore_axis_name="s",
                                num_cores=NC, num_subcores=NS),
    scratch_types=[
        pltpu.VMEM((NCHUNK, CHUNK), jnp.int32),
        pltpu.VMEM((NCHUNK, CHUNK), jnp.int32),
        pltpu.VMEM((BPW,), jnp.float32),
        pltpu.VMEM((BPW,), jnp.float32),
        pltpu.VMEM((BPW,), jnp.float32),
        pltpu.SemaphoreType.DMA,
    ],
)


def kernel(user_ids, item_ids, user_features, item_features, user_emb,
           item_emb, W_uf, b_uf, W_if, b_if, W_out, b_out):
    wout = W_out.reshape(1, 2 * D)
    pu, pi, f = _pv(wout, user_emb.T, item_emb.T,
                    user_features.T, item_features.T, W_uf, W_if,
                    b_uf.reshape(1, D), b_if.reshape(1, D),
                    b_out.reshape(1, 1))
    idxu = user_ids.reshape(NW * NCHUNK, CHUNK)
    idxi = item_ids.reshape(NW * NCHUNK, CHUNK)
    out = _sc_gather(pu, pi, f.reshape(B), idxu, idxi)
    return out.reshape(B, 1)


# speedup vs baseline: 9.4501x; 1.1066x over previous
import jax
import jax.numpy as jnp
from jax import lax
from jax.experimental import pallas as pl
from jax.experimental.pallas import tpu as pltpu
from jax.experimental.pallas import tpu_sc as plsc

B = 16384
D = 32
FD = 16
NROWS = 1000000
BLK = 32768
PADN = 1015808
GA = PADN // BLK

NC = 2
NS = 16
NW = NC * NS
BPW = B // NW
CHUNK = 128
NCHUNK = BPW // CHUNK


def _pv_body(wout, ut, it, uft, ift, wuf, wif, buf, bif, bo, pu, pi, f):
    f32 = jnp.float32
    wu = wout[:, :D]
    wi = wout[:, D:]
    pu[...] = jnp.dot(wu, ut[...], preferred_element_type=f32).reshape(BLK)
    pi[...] = jnp.dot(wi, it[...], preferred_element_type=f32).reshape(BLK)

    @pl.when(pl.program_id(0) == 0)
    def _():
        qu = lax.dot_general(wuf[...], wu, (((1,), (1,)), ((), ())),
                             preferred_element_type=f32)
        qi = lax.dot_general(wif[...], wi, (((1,), (1,)), ((), ())),
                             preferred_element_type=f32)
        fu = lax.dot_general(qu, uft[...], (((0,), (0,)), ((), ())),
                             preferred_element_type=f32)
        fi = lax.dot_general(qi, ift[...], (((0,), (0,)), ((), ())),
                             preferred_element_type=f32)
        const = (jnp.sum(buf[...] * wu) + jnp.sum(bif[...] * wi) + bo[0, 0])
        f[...] = fu + fi + const


_pv = pl.pallas_call(
    _pv_body,
    grid=(GA,),
    in_specs=[
        pl.BlockSpec((1, 2 * D), lambda j: (0, 0)),
        pl.BlockSpec((D, BLK), lambda j: (0, j)),
        pl.BlockSpec((D, BLK), lambda j: (0, j)),
        pl.BlockSpec((FD, B), lambda j: (0, 0)),
        pl.BlockSpec((FD, B), lambda j: (0, 0)),
        pl.BlockSpec((FD, D), lambda j: (0, 0)),
        pl.BlockSpec((FD, D), lambda j: (0, 0)),
        pl.BlockSpec((1, D), lambda j: (0, 0)),
        pl.BlockSpec((1, D), lambda j: (0, 0)),
        pl.BlockSpec((1, 1), lambda j: (0, 0)),
    ],
    out_specs=[
        pl.BlockSpec((BLK,), lambda j: (j,)),
        pl.BlockSpec((BLK,), lambda j: (j,)),
        pl.BlockSpec((1, B), lambda j: (0, 0)),
    ],
    out_shape=[
        jax.ShapeDtypeStruct((PADN,), jnp.float32),
        jax.ShapeDtypeStruct((PADN,), jnp.float32),
        jax.ShapeDtypeStruct((1, B), jnp.float32),
    ],
)


def _sc_body(pu, pi, f, idxu, idxi, out, idxvu, idxvi, valsu, valsi, fv,
             sem):
    wid = lax.axis_index("s") * NC + lax.axis_index("c")
    base = wid * BPW
    pltpu.sync_copy(idxu.at[pl.ds(wid * NCHUNK, NCHUNK)], idxvu)
    pltpu.sync_copy(idxi.at[pl.ds(wid * NCHUNK, NCHUNK)], idxvi)
    for j in range(NCHUNK):
        pltpu.async_copy(pu.at[idxvu.at[j]],
                         valsu.at[pl.ds(j * CHUNK, CHUNK)], sem)
    for j in range(NCHUNK):
        pltpu.async_copy(pi.at[idxvi.at[j]],
                         valsi.at[pl.ds(j * CHUNK, CHUNK)], sem)
    pltpu.sync_copy(f.at[pl.ds(base, BPW)], fv)
    for j in range(NCHUNK):
        pltpu.make_async_copy(pu.at[idxvu.at[j]],
                              valsu.at[pl.ds(j * CHUNK, CHUNK)], sem).wait()
    for j in range(NCHUNK):
        pltpu.make_async_copy(pi.at[idxvi.at[j]],
                              valsi.at[pl.ds(j * CHUNK, CHUNK)], sem).wait()
    valsu[...] = valsu[...] + valsi[...] + fv[...]
    pltpu.sync_copy(valsu, out.at[pl.ds(base, BPW)])


_sc_gather = pl.kernel(
    _sc_body,
    out_type=jax.ShapeDtypeStruct((B,), jnp.float32),
    mesh=plsc.VectorSubcoreMesh(core_axis_name="c", subcore_axis_name="s",
                                num_cores=NC, num_subcores=NS),
    scratch_types=[
        pltpu.VMEM((NCHUNK, CHUNK), jnp.int32),
        pltpu.VMEM((NCHUNK, CHUNK), jnp.int32),
        pltpu.VMEM((BPW,), jnp.float32),
        pltpu.VMEM((BPW,), jnp.float32),
        pltpu.VMEM((BPW,), jnp.float32),
        pltpu.SemaphoreType.DMA,
    ],
)


def kernel(user_ids, item_ids, user_features, item_features, user_emb,
           item_emb, W_uf, b_uf, W_if, b_if, W_out, b_out):
    wout = W_out.reshape(1, 2 * D)
    pu, pi, f = _pv(wout, user_emb.T, item_emb.T,
                    user_features.T, item_features.T, W_uf, W_if,
                    b_uf.reshape(1, D), b_if.reshape(1, D),
                    b_out.reshape(1, 1))
    idxu = user_ids.reshape(NW * NCHUNK, CHUNK)
    idxi = item_ids.reshape(NW * NCHUNK, CHUNK)
    out = _sc_gather(pu, pi, f.reshape(B), idxu, idxi)
    return out.reshape(B, 1)
